# baseline (speedup 1.0000x reference)
import jax
import jax.numpy as jnp
from jax.experimental import pallas as pl
from jax.experimental.pallas import tpu as pltpu

_BLOCK_ROWS = 512


def _add_row_body(idx_ref, emb_ref, x_hbm, o_hbm, ibuf, obuf, in_sem, out_sem):
    n = x_hbm.shape[0] // _BLOCK_ROWS
    row = emb_ref[idx_ref[0], :]

    def in_copy(i):
        return pltpu.make_async_copy(
            x_hbm.at[pl.ds(i * _BLOCK_ROWS, _BLOCK_ROWS)],
            ibuf.at[i % 2],
            in_sem.at[i % 2],
        )

    def out_copy(i):
        return pltpu.make_async_copy(
            obuf.at[i % 2],
            o_hbm.at[pl.ds(i * _BLOCK_ROWS, _BLOCK_ROWS)],
            out_sem.at[i % 2],
        )

    in_copy(0).start()
    in_copy(1).start()
    for i in range(n):
        in_copy(i).wait()
        if i >= 2:
            out_copy(i - 2).wait()
        obuf[i % 2] = ibuf[i % 2] + row[None, :]
        out_copy(i).start()
        if i + 2 < n:
            in_copy(i + 2).start()
    out_copy(n - 2).wait()
    out_copy(n - 1).wait()


def kernel(x, rate_id, rate_embedding):
    b, s, d = x.shape
    rows = b * s
    x2 = x.reshape(rows, d)
    idx = jnp.asarray([rate_id], dtype=jnp.int32)
    out = pl.pallas_call(
        _add_row_body,
        in_specs=[
            pl.BlockSpec(memory_space=pltpu.SMEM),
            pl.BlockSpec(memory_space=pltpu.VMEM),
            pl.BlockSpec(memory_space=pl.ANY),
        ],
        out_specs=pl.BlockSpec(memory_space=pl.ANY),
        out_shape=jax.ShapeDtypeStruct((rows, d), x.dtype),
        scratch_shapes=[
            pltpu.VMEM((2, _BLOCK_ROWS, d), x.dtype),
            pltpu.VMEM((2, _BLOCK_ROWS, d), x.dtype),
            pltpu.SemaphoreType.DMA((2,)),
            pltpu.SemaphoreType.DMA((2,)),
        ],
    )(idx, rate_embedding, x2)
    return out.reshape(b, s, d)


# speedup vs baseline: 1.1002x; 1.1002x over previous
import jax
import jax.numpy as jnp
from jax.experimental import pallas as pl
from jax.experimental.pallas import tpu as pltpu

_BLOCK_ROWS = 512
_SLOTS = 4


def _add_row_body(idx_ref, emb_ref, x_hbm, o_hbm, ibuf, obuf, in_sem, out_sem):
    n = x_hbm.shape[0] // _BLOCK_ROWS
    row = emb_ref[idx_ref[0], :]

    def in_copy(i):
        return pltpu.make_async_copy(
            x_hbm.at[pl.ds(i * _BLOCK_ROWS, _BLOCK_ROWS)],
            ibuf.at[i % _SLOTS],
            in_sem.at[i % _SLOTS],
        )

    def out_copy(i):
        return pltpu.make_async_copy(
            obuf.at[i % _SLOTS],
            o_hbm.at[pl.ds(i * _BLOCK_ROWS, _BLOCK_ROWS)],
            out_sem.at[i % _SLOTS],
        )

    for i in range(min(_SLOTS, n)):
        in_copy(i).start()
    for i in range(n):
        in_copy(i).wait()
        if i >= _SLOTS:
            out_copy(i - _SLOTS).wait()
        obuf[i % _SLOTS] = ibuf[i % _SLOTS] + row[None, :]
        out_copy(i).start()
        if i + _SLOTS < n:
            in_copy(i + _SLOTS).start()
    for i in range(max(n - _SLOTS, 0), n):
        out_copy(i).wait()


def kernel(x, rate_id, rate_embedding):
    b, s, d = x.shape
    rows = b * s
    x2 = x.reshape(rows, d)
    idx = jnp.asarray([rate_id], dtype=jnp.int32)
    out = pl.pallas_call(
        _add_row_body,
        in_specs=[
            pl.BlockSpec(memory_space=pltpu.SMEM),
            pl.BlockSpec(memory_space=pltpu.VMEM),
            pl.BlockSpec(memory_space=pl.ANY),
        ],
        out_specs=pl.BlockSpec(memory_space=pl.ANY),
        out_shape=jax.ShapeDtypeStruct((rows, d), x.dtype),
        scratch_shapes=[
            pltpu.VMEM((_SLOTS, _BLOCK_ROWS, d), x.dtype),
            pltpu.VMEM((_SLOTS, _BLOCK_ROWS, d), x.dtype),
            pltpu.SemaphoreType.DMA((_SLOTS,)),
            pltpu.SemaphoreType.DMA((_SLOTS,)),
        ],
    )(idx, rate_embedding, x2)
    return out.reshape(b, s, d)


# speedup vs baseline: 1.1040x; 1.0035x over previous
import jax
import jax.numpy as jnp
from jax.experimental import pallas as pl
from jax.experimental.pallas import tpu as pltpu

_BLOCK_ROWS = 1024
_SLOTS = 4


def _add_row_body(idx_ref, emb_ref, x_hbm, o_hbm, ibuf, obuf, in_sem, out_sem):
    n = x_hbm.shape[0] // _BLOCK_ROWS
    row = emb_ref[idx_ref[0], :]

    def in_copy(i):
        return pltpu.make_async_copy(
            x_hbm.at[pl.ds(i * _BLOCK_ROWS, _BLOCK_ROWS)],
            ibuf.at[i % _SLOTS],
            in_sem.at[i % _SLOTS],
        )

    def out_copy(i):
        return pltpu.make_async_copy(
            obuf.at[i % _SLOTS],
            o_hbm.at[pl.ds(i * _BLOCK_ROWS, _BLOCK_ROWS)],
            out_sem.at[i % _SLOTS],
        )

    for i in range(min(_SLOTS, n)):
        in_copy(i).start()
    for i in range(n):
        in_copy(i).wait()
        if i >= _SLOTS:
            out_copy(i - _SLOTS).wait()
        obuf[i % _SLOTS] = ibuf[i % _SLOTS] + row[None, :]
        out_copy(i).start()
        if i + _SLOTS < n:
            in_copy(i + _SLOTS).start()
    for i in range(max(n - _SLOTS, 0), n):
        out_copy(i).wait()


def kernel(x, rate_id, rate_embedding):
    b, s, d = x.shape
    rows = b * s
    x2 = x.reshape(rows, d)
    idx = jnp.asarray([rate_id], dtype=jnp.int32)
    out = pl.pallas_call(
        _add_row_body,
        in_specs=[
            pl.BlockSpec(memory_space=pltpu.SMEM),
            pl.BlockSpec(memory_space=pltpu.VMEM),
            pl.BlockSpec(memory_space=pl.ANY),
        ],
        out_specs=pl.BlockSpec(memory_space=pl.ANY),
        out_shape=jax.ShapeDtypeStruct((rows, d), x.dtype),
        scratch_shapes=[
            pltpu.VMEM((_SLOTS, _BLOCK_ROWS, d), x.dtype),
            pltpu.VMEM((_SLOTS, _BLOCK_ROWS, d), x.dtype),
            pltpu.SemaphoreType.DMA((_SLOTS,)),
            pltpu.SemaphoreType.DMA((_SLOTS,)),
        ],
    )(idx, rate_embedding, x2)
    return out.reshape(b, s, d)


# speedup vs baseline: 1.1113x; 1.0066x over previous
import jax
import jax.numpy as jnp
from jax.experimental import pallas as pl
from jax.experimental.pallas import tpu as pltpu

_BLOCK_ROWS = 2048
_SLOTS = 3


def _add_row_body(idx_ref, emb_ref, x_hbm, o_hbm, ibuf, obuf, in_sem, out_sem):
    n = x_hbm.shape[0] // _BLOCK_ROWS
    row = emb_ref[idx_ref[0], :]

    def in_copy(i):
        return pltpu.make_async_copy(
            x_hbm.at[pl.ds(i * _BLOCK_ROWS, _BLOCK_ROWS)],
            ibuf.at[i % _SLOTS],
            in_sem.at[i % _SLOTS],
        )

    def out_copy(i):
        return pltpu.make_async_copy(
            obuf.at[i % _SLOTS],
            o_hbm.at[pl.ds(i * _BLOCK_ROWS, _BLOCK_ROWS)],
            out_sem.at[i % _SLOTS],
        )

    for i in range(min(_SLOTS, n)):
        in_copy(i).start()
    for i in range(n):
        in_copy(i).wait()
        if i >= _SLOTS:
            out_copy(i - _SLOTS).wait()
        obuf[i % _SLOTS] = ibuf[i % _SLOTS] + row[None, :]
        out_copy(i).start()
        if i + _SLOTS < n:
            in_copy(i + _SLOTS).start()
    for i in range(max(n - _SLOTS, 0), n):
        out_copy(i).wait()


def kernel(x, rate_id, rate_embedding):
    b, s, d = x.shape
    rows = b * s
    x2 = x.reshape(rows, d)
    idx = jnp.asarray([rate_id], dtype=jnp.int32)
    out = pl.pallas_call(
        _add_row_body,
        in_specs=[
            pl.BlockSpec(memory_space=pltpu.SMEM),
            pl.BlockSpec(memory_space=pltpu.VMEM),
            pl.BlockSpec(memory_space=pl.ANY),
        ],
        out_specs=pl.BlockSpec(memory_space=pl.ANY),
        out_shape=jax.ShapeDtypeStruct((rows, d), x.dtype),
        scratch_shapes=[
            pltpu.VMEM((_SLOTS, _BLOCK_ROWS, d), x.dtype),
            pltpu.VMEM((_SLOTS, _BLOCK_ROWS, d), x.dtype),
            pltpu.SemaphoreType.DMA((_SLOTS,)),
            pltpu.SemaphoreType.DMA((_SLOTS,)),
        ],
    )(idx, rate_embedding, x2)
    return out.reshape(b, s, d)
